# baseline (speedup 1.0000x reference)
import functools

import jax
import jax.numpy as jnp
from jax import lax
from jax.experimental import pallas as pl
from jax.experimental.pallas import tpu as pltpu
from jax.experimental.pallas import tpu_sc as plsc

N = 10000
E = 320000
F = 512
H = 128
NLAYERS = 3

NC = 2
NS = 16
NW = NC * NS
EPW = E // NW
C = 96
NCHUNK = -(-EPW // C)
EPAD = NCHUNK * C
ND = 3
NI = 4
NPAD = N + 8
ZR = 624
ZREM = NPAD - NS * ZR
WREM = N - NS * ZR
HV = H // 16
TPAD = 104
DUMT = 100


def _sc_agg_body(x_hbm, edata_hbm, emb_hbm, zero_hbm, out_hbm,
                 idx_v, buf_v, emb_sh, agg_sh,
                 sem_i, sem_e, sem_x, sem_s):
    c = lax.axis_index("c")
    s = lax.axis_index("s")
    w = c * NS + s

    def start_idx(k):
        pltpu.async_copy(edata_hbm.at[w, k], idx_v.at[lax.rem(k, NI)], sem_i)

    def wait_idx():
        pltpu.make_async_copy(edata_hbm.at[0, 0], idx_v.at[0], sem_i).wait()

    def start_e(k):
        pltpu.async_copy(emb_sh.at[idx_v.at[lax.rem(k, NI), 0]],
                         buf_v.at[lax.rem(k, ND)], sem_e)

    def start_x(k):
        pltpu.async_copy(x_hbm.at[idx_v.at[lax.rem(k, NI), 1]],
                         buf_v.at[lax.rem(k, ND)], sem_x, add=True)

    def start_scat(k):
        pltpu.async_copy(buf_v.at[lax.rem(k, ND)],
                         agg_sh.at[idx_v.at[lax.rem(k, NI), 2]], sem_s,
                         add=True)

    def drain(sem):
        pltpu.make_async_copy(x_hbm.at[pl.ds(0, C)], buf_v.at[0], sem).wait()

    def relu_chunk(k):
        q = lax.rem(k, ND)

        def relu_row(r, carry):
            for j in range(HV):
                v = buf_v[q, r, pl.ds(j * 16, 16)]
                buf_v[q, r, pl.ds(j * 16, 16)] = jnp.maximum(v, 0.0)
            return carry
        lax.fori_loop(0, C, relu_row, 0)

    pltpu.sync_copy(edata_hbm.at[w, 0], idx_v.at[0])
    start_idx(1)
    start_idx(2)

    @pl.when(s == 0)
    def _load_emb():
        pltpu.sync_copy(emb_hbm, emb_sh)

    zbase = pl.multiple_of(s * ZR, 8)
    pltpu.sync_copy(zero_hbm.at[pl.ds(zbase, ZR)],
                    agg_sh.at[pl.ds(zbase, ZR)])

    @pl.when(s == NS - 1)
    def _zero_rem():
        pltpu.sync_copy(zero_hbm.at[pl.ds(NS * ZR, ZREM)],
                        agg_sh.at[pl.ds(NS * ZR, ZREM)])

    plsc.subcore_barrier()

    start_e(0)
    drain(sem_e)
    start_x(0)
    wait_idx()
    start_e(1)

    def slot(k, carry):
        drain(sem_x)

        @pl.when(k + 1 < NCHUNK)
        def _next_x():
            drain(sem_e)
            start_x(k + 1)

        relu_chunk(k)

        @pl.when(k > 0)
        def _prev_scat():
            drain(sem_s)

        start_scat(k)

        @pl.when(k + 2 < NCHUNK)
        def _next_e():
            wait_idx()
            start_e(k + 2)

        @pl.when(k + 3 < NCHUNK)
        def _next_idx():
            start_idx(k + 3)
        return carry

    lax.fori_loop(0, NCHUNK, slot, 0)
    drain(sem_s)
    plsc.subcore_barrier()

    wbase = pl.multiple_of(s * ZR, 8)
    pltpu.sync_copy(agg_sh.at[pl.ds(wbase, ZR)],
                    out_hbm.at[c, pl.ds(wbase, ZR)])

    @pl.when(s == NS - 1)
    def _wb_rem():
        pltpu.sync_copy(agg_sh.at[pl.ds(NS * ZR, WREM)],
                        out_hbm.at[c, pl.ds(NS * ZR, WREM)])


_sc_agg = pl.kernel(
    _sc_agg_body,
    out_type=jax.ShapeDtypeStruct((NC, N, H), jnp.float32),
    mesh=plsc.VectorSubcoreMesh(core_axis_name="c", subcore_axis_name="s"),
    scratch_types=[
        pltpu.VMEM((NI, 3, C), jnp.int32),
        pltpu.VMEM((ND, C, H), jnp.float32),
        pltpu.VMEM_SHARED((TPAD, H), jnp.float32),
        pltpu.VMEM_SHARED((NPAD, H), jnp.float32),
        pltpu.SemaphoreType.DMA,
        pltpu.SemaphoreType.DMA,
        pltpu.SemaphoreType.DMA,
        pltpu.SemaphoreType.DMA,
    ],
)


BR = 1000


def _proj_body(p_ref, wp_ref, bp_ref, o_ref):
    o_ref[...] = jnp.dot(p_ref[...], wp_ref[...],
                         preferred_element_type=jnp.float32) + bp_ref[...]


_proj = pl.pallas_call(
    _proj_body,
    grid=(N // BR,),
    in_specs=[
        pl.BlockSpec((BR, F), lambda i: (i, 0)),
        pl.BlockSpec((F, H), lambda i: (0, 0)),
        pl.BlockSpec((1, H), lambda i: (0, 0)),
    ],
    out_specs=pl.BlockSpec((BR, H), lambda i: (i, 0)),
    out_shape=jax.ShapeDtypeStruct((N, H), jnp.float32),
)


def _mlp_body(x_ref, agg_ref, w1_ref, b1_ref, w2_ref, b2_ref, g_ref, be_ref,
              o_ref, *, last):
    x = x_ref[...]
    h0 = x + agg_ref[0] + agg_ref[1]
    t = jnp.maximum(jnp.dot(h0, w1_ref[...],
                            preferred_element_type=jnp.float32) + b1_ref[...],
                    0.0)
    h = jnp.dot(t, w2_ref[...],
                preferred_element_type=jnp.float32) + b2_ref[...] + x
    if last:
        mu = jnp.mean(h, axis=-1, keepdims=True)
        var = jnp.mean((h - mu) ** 2, axis=-1, keepdims=True)
        h = (h - mu) * lax.rsqrt(var + 1e-5) * g_ref[...] + be_ref[...]
    o_ref[...] = h


def _make_mlp(last):
    return pl.pallas_call(
        functools.partial(_mlp_body, last=last),
        grid=(N // BR,),
        in_specs=[
            pl.BlockSpec((BR, H), lambda i: (i, 0)),
            pl.BlockSpec((NC, BR, H), lambda i: (0, i, 0)),
            pl.BlockSpec((H, H), lambda i: (0, 0)),
            pl.BlockSpec((1, H), lambda i: (0, 0)),
            pl.BlockSpec((H, H), lambda i: (0, 0)),
            pl.BlockSpec((1, H), lambda i: (0, 0)),
            pl.BlockSpec((1, H), lambda i: (0, 0)),
            pl.BlockSpec((1, H), lambda i: (0, 0)),
        ],
        out_specs=pl.BlockSpec((BR, H), lambda i: (i, 0)),
        out_shape=jax.ShapeDtypeStruct((N, H), jnp.float32),
    )


_mlp_mid = _make_mlp(False)
_mlp_last = _make_mlp(True)


def kernel(peptide_feature, edge_index, edge_attr, Wp, bp, W1, b1, W2, b2,
           emb_table, gamma, beta):
    src = edge_index[0]
    dst = edge_index[1]
    tt = edge_attr[:, 0]
    pads = (jnp.full((NW, EPAD - EPW), DUMT, jnp.int32),
            jnp.zeros((NW, EPAD - EPW), jnp.int32),
            jnp.full((NW, EPAD - EPW), N, jnp.int32))
    edata = jnp.stack([jnp.concatenate([a.reshape(NW, EPW), p], axis=1)
                       for a, p in zip((tt, src, dst), pads)], axis=1)
    edata = edata.reshape(NW, 3, NCHUNK, C).transpose(0, 2, 1, 3)
    emb_p = jnp.zeros((TPAD, H), jnp.float32).at[:100].set(emb_table)
    zeros = jnp.zeros((NPAD, H), jnp.float32)
    bp2 = bp.reshape(1, H)
    g2 = gamma.reshape(1, H)
    be2 = beta.reshape(1, H)

    x = _proj(peptide_feature, Wp, bp2)
    for i in range(NLAYERS):
        agg = _sc_agg(x, edata, emb_p, zeros)
        mlp = _mlp_last if i == NLAYERS - 1 else _mlp_mid
        x = mlp(x, agg, W1[i], b1[i].reshape(1, H), W2[i],
                b2[i].reshape(1, H), g2, be2)
    return x


# speedup vs baseline: 1.5377x; 1.5377x over previous
import functools

import jax
import jax.numpy as jnp
from jax import lax
from jax.experimental import pallas as pl
from jax.experimental.pallas import tpu as pltpu
from jax.experimental.pallas import tpu_sc as plsc

N = 10000
E = 320000
F = 512
H = 128
NLAYERS = 3

NC = 2
NS = 16
NW = NC * NS
EPW = E // NW
C = 96
NCHUNK = -(-EPW // C)
EPAD = NCHUNK * C
ND = 3
NI = 4
NPAD = N + 16
ZR = 624
ZREM = NPAD - NS * ZR
WREM = N - NS * ZR
HV = H // 16
TPAD = 104
DUMT = 100


def _sc_agg_body(x_hbm, edata_hbm, emb_hbm, zero_hbm, out_hbm,
                 idxa_v, idxb_v, bufa_v, bufb_v, emb_sh, agg_sh,
                 sem_ia, sem_ib, sem_ea, sem_eb, sem_xa, sem_xb):
    c = lax.axis_index("c")
    s = lax.axis_index("s")
    w = c * NS + s

    def start_idx(k, idx, sem):
        pltpu.async_copy(edata_hbm.at[w, k], idx, sem)

    def wait_idx(idx, sem):
        pltpu.make_async_copy(edata_hbm.at[0, 0], idx, sem).wait()

    def start_e(idx, buf, sem):
        pltpu.async_copy(emb_sh.at[idx.at[0]], buf, sem)

    def start_x(idx, buf, sem):
        pltpu.async_copy(x_hbm.at[idx.at[1]], buf, sem, add=True)

    def relu_buf(buf):
        def relu_row(r, carry):
            for j in range(HV):
                v = buf[r, pl.ds(j * 16, 16)]
                buf[r, pl.ds(j * 16, 16)] = jnp.maximum(v, 0.0)
            return carry
        lax.fori_loop(0, C, relu_row, 0)

    def scat(idx, buf):
        pltpu.sync_copy(buf, agg_sh.at[idx.at[2]], add=True)

    def wait_buf(buf, sem):
        pltpu.make_async_copy(x_hbm.at[pl.ds(0, C)], buf, sem).wait()

    pltpu.sync_copy(edata_hbm.at[w, 0], idxa_v)
    start_idx(1, idxb_v, sem_ib)

    @pl.when(s == 0)
    def _load_emb():
        pltpu.sync_copy(emb_hbm, emb_sh)

    zbase = pl.multiple_of(s * ZR, 8)
    pltpu.sync_copy(zero_hbm.at[pl.ds(zbase, ZR)],
                    agg_sh.at[pl.ds(zbase, ZR)])

    @pl.when(s == NS - 1)
    def _zero_rem():
        pltpu.sync_copy(zero_hbm.at[pl.ds(NS * ZR, ZREM)],
                        agg_sh.at[pl.ds(NS * ZR, ZREM)])

    plsc.subcore_barrier()
    start_e(idxa_v, bufa_v, sem_ea)

    def body(m, carry):
        k2 = 2 * m + 2
        k3 = 2 * m + 3
        wait_buf(bufa_v, sem_ea)
        start_x(idxa_v, bufa_v, sem_xa)
        wait_idx(idxb_v, sem_ib)
        start_e(idxb_v, bufb_v, sem_eb)
        wait_buf(bufa_v, sem_xa)
        relu_buf(bufa_v)
        scat(idxa_v, bufa_v)
        start_idx(k2, idxa_v, sem_ia)
        wait_buf(bufb_v, sem_eb)
        start_x(idxb_v, bufb_v, sem_xb)
        wait_idx(idxa_v, sem_ia)
        start_e(idxa_v, bufa_v, sem_ea)
        wait_buf(bufb_v, sem_xb)
        relu_buf(bufb_v)
        scat(idxb_v, bufb_v)

        @pl.when(k3 < NCHUNK)
        def _pf():
            start_idx(k3, idxb_v, sem_ib)
        return carry

    lax.fori_loop(0, (NCHUNK - 1) // 2, body, 0)

    wait_buf(bufa_v, sem_ea)
    start_x(idxa_v, bufa_v, sem_xa)
    wait_buf(bufa_v, sem_xa)
    relu_buf(bufa_v)
    scat(idxa_v, bufa_v)

    plsc.subcore_barrier()

    wbase = pl.multiple_of(s * ZR, 8)
    pltpu.sync_copy(agg_sh.at[pl.ds(wbase, ZR)],
                    out_hbm.at[c, pl.ds(wbase, ZR)])

    @pl.when(s == NS - 1)
    def _wb_rem():
        pltpu.sync_copy(agg_sh.at[pl.ds(NS * ZR, WREM)],
                        out_hbm.at[c, pl.ds(NS * ZR, WREM)])


_sc_agg = pl.kernel(
    _sc_agg_body,
    out_type=jax.ShapeDtypeStruct((NC, N, H), jnp.float32),
    mesh=plsc.VectorSubcoreMesh(core_axis_name="c", subcore_axis_name="s"),
    scratch_types=[
        pltpu.VMEM((3, C), jnp.int32),
        pltpu.VMEM((3, C), jnp.int32),
        pltpu.VMEM((C, H), jnp.float32),
        pltpu.VMEM((C, H), jnp.float32),
        pltpu.VMEM_SHARED((TPAD, H), jnp.float32),
        pltpu.VMEM_SHARED((NPAD, H), jnp.float32),
        pltpu.SemaphoreType.DMA,
        pltpu.SemaphoreType.DMA,
        pltpu.SemaphoreType.DMA,
        pltpu.SemaphoreType.DMA,
        pltpu.SemaphoreType.DMA,
        pltpu.SemaphoreType.DMA,
    ],
)


BR = 1000


def _proj_body(p_ref, wp_ref, bp_ref, o_ref):
    o_ref[...] = jnp.dot(p_ref[...], wp_ref[...],
                         preferred_element_type=jnp.float32) + bp_ref[...]


_proj = pl.pallas_call(
    _proj_body,
    grid=(N // BR,),
    in_specs=[
        pl.BlockSpec((BR, F), lambda i: (i, 0)),
        pl.BlockSpec((F, H), lambda i: (0, 0)),
        pl.BlockSpec((1, H), lambda i: (0, 0)),
    ],
    out_specs=pl.BlockSpec((BR, H), lambda i: (i, 0)),
    out_shape=jax.ShapeDtypeStruct((N, H), jnp.float32),
)


def _mlp_body(x_ref, agg_ref, w1_ref, b1_ref, w2_ref, b2_ref, g_ref, be_ref,
              o_ref, *, last):
    x = x_ref[...]
    h0 = x + agg_ref[0] + agg_ref[1]
    t = jnp.maximum(jnp.dot(h0, w1_ref[...],
                            preferred_element_type=jnp.float32) + b1_ref[...],
                    0.0)
    h = jnp.dot(t, w2_ref[...],
                preferred_element_type=jnp.float32) + b2_ref[...] + x
    if last:
        mu = jnp.mean(h, axis=-1, keepdims=True)
        var = jnp.mean((h - mu) ** 2, axis=-1, keepdims=True)
        h = (h - mu) * lax.rsqrt(var + 1e-5) * g_ref[...] + be_ref[...]
    o_ref[...] = h


def _make_mlp(last):
    return pl.pallas_call(
        functools.partial(_mlp_body, last=last),
        grid=(N // BR,),
        in_specs=[
            pl.BlockSpec((BR, H), lambda i: (i, 0)),
            pl.BlockSpec((NC, BR, H), lambda i: (0, i, 0)),
            pl.BlockSpec((H, H), lambda i: (0, 0)),
            pl.BlockSpec((1, H), lambda i: (0, 0)),
            pl.BlockSpec((H, H), lambda i: (0, 0)),
            pl.BlockSpec((1, H), lambda i: (0, 0)),
            pl.BlockSpec((1, H), lambda i: (0, 0)),
            pl.BlockSpec((1, H), lambda i: (0, 0)),
        ],
        out_specs=pl.BlockSpec((BR, H), lambda i: (i, 0)),
        out_shape=jax.ShapeDtypeStruct((N, H), jnp.float32),
    )


_mlp_mid = _make_mlp(False)
_mlp_last = _make_mlp(True)


def kernel(peptide_feature, edge_index, edge_attr, Wp, bp, W1, b1, W2, b2,
           emb_table, gamma, beta):
    src = edge_index[0]
    dst = edge_index[1]
    tt = edge_attr[:, 0]
    trash = N + (jnp.arange(NW, dtype=jnp.int32) % 16)
    pads = (jnp.full((NW, EPAD - EPW), DUMT, jnp.int32),
            jnp.zeros((NW, EPAD - EPW), jnp.int32),
            jnp.broadcast_to(trash[:, None], (NW, EPAD - EPW)))
    edata = jnp.stack([jnp.concatenate([a.reshape(NW, EPW), p], axis=1)
                       for a, p in zip((tt, src, dst), pads)], axis=1)
    edata = edata.reshape(NW, 3, NCHUNK, C).transpose(0, 2, 1, 3)
    emb_p = jnp.zeros((TPAD, H), jnp.float32).at[:100].set(emb_table)
    zeros = jnp.zeros((NPAD, H), jnp.float32)
    bp2 = bp.reshape(1, H)
    g2 = gamma.reshape(1, H)
    be2 = beta.reshape(1, H)

    x = _proj(peptide_feature, Wp, bp2)
    for i in range(NLAYERS):
        agg = _sc_agg(x, edata, emb_p, zeros)
        mlp = _mlp_last if i == NLAYERS - 1 else _mlp_mid
        x = mlp(x, agg, W1[i], b1[i].reshape(1, H), W2[i],
                b2[i].reshape(1, H), g2, be2)
    return x


# speedup vs baseline: 2.4036x; 1.5632x over previous
import functools

import jax
import jax.numpy as jnp
from jax import lax
from jax.experimental import pallas as pl
from jax.experimental.pallas import tpu as pltpu
from jax.experimental.pallas import tpu_sc as plsc

N = 10000
E = 320000
F = 512
H = 128
NLAYERS = 3

NC = 2
NS = 16
NW = NC * NS
EPW = E // NW
C = 80
NCHUNK = EPW // C
UNROLL = 6
NB = 3
ZR = 624
ZREM = N - NS * ZR
HV = H // 16
TPAD = 104


def _sc_agg_body(x_hbm, edata_hbm, emb_hbm, zero_hbm, out_hbm,
                 i0_v, i1_v, i2_v, i3_v, i4_v, i5_v, b0_v, b1_v, b2_v,
                 emb_sh, agg_sh, sem_i, sem_e, sem_x, sem_s):
    c = lax.axis_index("c")
    s = lax.axis_index("s")
    w = c * NS + s
    idxs = (i0_v, i1_v, i2_v, i3_v, i4_v, i5_v)
    bufs = (b0_v, b1_v, b2_v)

    def start_idx(k, j):
        pltpu.async_copy(edata_hbm.at[w, k], idxs[j % UNROLL], sem_i)

    def start_e(j):
        pltpu.async_copy(emb_sh.at[idxs[j % UNROLL].at[0]], bufs[j % NB],
                         sem_e)

    def start_x(j):
        pltpu.async_copy(x_hbm.at[idxs[j % UNROLL].at[1]], bufs[j % NB],
                         sem_x, add=True)

    def start_scat(j):
        pltpu.async_copy(bufs[j % NB], agg_sh.at[idxs[j % UNROLL].at[2]],
                         sem_s, add=True)

    def drain(sem):
        pltpu.make_async_copy(x_hbm.at[pl.ds(0, C)], b0_v, sem).wait()

    def drain_idx():
        pltpu.make_async_copy(edata_hbm.at[0, 0], i0_v, sem_i).wait()

    def relu_buf(buf):
        def relu_row(r, carry):
            for j in range(HV):
                v = buf[r, pl.ds(j * 16, 16)]
                buf[r, pl.ds(j * 16, 16)] = jnp.maximum(v, 0.0)
            return carry
        lax.fori_loop(0, C, relu_row, 0)

    def slot(k, j):
        drain(sem_x)

        @pl.when(k + 1 < NCHUNK)
        def _next_x():
            drain(sem_e)
            start_x(j + 1)

        relu_buf(bufs[j % NB])

        @pl.when(k > 0)
        def _prev_scat():
            drain(sem_s)

        start_scat(j)

        @pl.when(k + 2 < NCHUNK)
        def _next_e():
            drain_idx()
            start_e(j + 2)

        @pl.when(k + 3 < NCHUNK)
        def _next_idx():
            start_idx(k + 3, j + 3)

    pltpu.sync_copy(edata_hbm.at[w, 0], i0_v)
    start_idx(1, 1)
    start_idx(2, 2)

    @pl.when(s == 0)
    def _load_emb():
        pltpu.sync_copy(emb_hbm, emb_sh)

    zbase = pl.multiple_of(s * ZR, 8)
    pltpu.sync_copy(zero_hbm.at[pl.ds(zbase, ZR)],
                    agg_sh.at[pl.ds(zbase, ZR)])

    @pl.when(s == NS - 1)
    def _zero_rem():
        pltpu.sync_copy(zero_hbm.at[pl.ds(NS * ZR, ZREM)],
                        agg_sh.at[pl.ds(NS * ZR, ZREM)])

    plsc.subcore_barrier()

    start_e(0)
    drain(sem_e)
    start_x(0)
    drain_idx()
    start_e(1)

    nmain = NCHUNK // UNROLL

    def body(m, carry):
        k0 = m * UNROLL
        for j in range(UNROLL):
            slot(k0 + j, j)
        return carry

    lax.fori_loop(0, nmain, body, 0)
    for j in range(NCHUNK - nmain * UNROLL):
        slot(nmain * UNROLL + j, j)

    drain(sem_s)
    plsc.subcore_barrier()

    wbase = pl.multiple_of(s * ZR, 8)
    pltpu.sync_copy(agg_sh.at[pl.ds(wbase, ZR)],
                    out_hbm.at[c, pl.ds(wbase, ZR)])

    @pl.when(s == NS - 1)
    def _wb_rem():
        pltpu.sync_copy(agg_sh.at[pl.ds(NS * ZR, ZREM)],
                        out_hbm.at[c, pl.ds(NS * ZR, ZREM)])


_sc_agg = pl.kernel(
    _sc_agg_body,
    out_type=jax.ShapeDtypeStruct((NC, N, H), jnp.float32),
    mesh=plsc.VectorSubcoreMesh(core_axis_name="c", subcore_axis_name="s"),
    scratch_types=[
        pltpu.VMEM((3, C), jnp.int32),
        pltpu.VMEM((3, C), jnp.int32),
        pltpu.VMEM((3, C), jnp.int32),
        pltpu.VMEM((3, C), jnp.int32),
        pltpu.VMEM((3, C), jnp.int32),
        pltpu.VMEM((3, C), jnp.int32),
        pltpu.VMEM((C, H), jnp.float32),
        pltpu.VMEM((C, H), jnp.float32),
        pltpu.VMEM((C, H), jnp.float32),
        pltpu.VMEM_SHARED((TPAD, H), jnp.float32),
        pltpu.VMEM_SHARED((N, H), jnp.float32),
        pltpu.SemaphoreType.DMA,
        pltpu.SemaphoreType.DMA,
        pltpu.SemaphoreType.DMA,
        pltpu.SemaphoreType.DMA,
    ],
)


BR = 1000


def _proj_body(p_ref, wp_ref, bp_ref, o_ref):
    o_ref[...] = jnp.dot(p_ref[...], wp_ref[...],
                         preferred_element_type=jnp.float32) + bp_ref[...]


_proj = pl.pallas_call(
    _proj_body,
    grid=(N // BR,),
    in_specs=[
        pl.BlockSpec((BR, F), lambda i: (i, 0)),
        pl.BlockSpec((F, H), lambda i: (0, 0)),
        pl.BlockSpec((1, H), lambda i: (0, 0)),
    ],
    out_specs=pl.BlockSpec((BR, H), lambda i: (i, 0)),
    out_shape=jax.ShapeDtypeStruct((N, H), jnp.float32),
)


def _mlp_body(x_ref, agg_ref, w1_ref, b1_ref, w2_ref, b2_ref, g_ref, be_ref,
              o_ref, *, last):
    x = x_ref[...]
    h0 = x + agg_ref[0] + agg_ref[1]
    t = jnp.maximum(jnp.dot(h0, w1_ref[...],
                            preferred_element_type=jnp.float32) + b1_ref[...],
                    0.0)
    h = jnp.dot(t, w2_ref[...],
                preferred_element_type=jnp.float32) + b2_ref[...] + x
    if last:
        mu = jnp.mean(h, axis=-1, keepdims=True)
        var = jnp.mean((h - mu) ** 2, axis=-1, keepdims=True)
        h = (h - mu) * lax.rsqrt(var + 1e-5) * g_ref[...] + be_ref[...]
    o_ref[...] = h


def _make_mlp(last):
    return pl.pallas_call(
        functools.partial(_mlp_body, last=last),
        grid=(N // BR,),
        in_specs=[
            pl.BlockSpec((BR, H), lambda i: (i, 0)),
            pl.BlockSpec((NC, BR, H), lambda i: (0, i, 0)),
            pl.BlockSpec((H, H), lambda i: (0, 0)),
            pl.BlockSpec((1, H), lambda i: (0, 0)),
            pl.BlockSpec((H, H), lambda i: (0, 0)),
            pl.BlockSpec((1, H), lambda i: (0, 0)),
            pl.BlockSpec((1, H), lambda i: (0, 0)),
            pl.BlockSpec((1, H), lambda i: (0, 0)),
        ],
        out_specs=pl.BlockSpec((BR, H), lambda i: (i, 0)),
        out_shape=jax.ShapeDtypeStruct((N, H), jnp.float32),
    )


_mlp_mid = _make_mlp(False)
_mlp_last = _make_mlp(True)


def kernel(peptide_feature, edge_index, edge_attr, Wp, bp, W1, b1, W2, b2,
           emb_table, gamma, beta):
    src = edge_index[0]
    dst = edge_index[1]
    tt = edge_attr[:, 0]
    edata = jnp.stack([tt, src, dst]).reshape(3, NW, NCHUNK, C)
    edata = edata.transpose(1, 2, 0, 3)
    emb_p = jnp.zeros((TPAD, H), jnp.float32).at[:100].set(emb_table)
    zeros = jnp.zeros((N, H), jnp.float32)
    bp2 = bp.reshape(1, H)
    g2 = gamma.reshape(1, H)
    be2 = beta.reshape(1, H)

    x = _proj(peptide_feature, Wp, bp2)
    for i in range(NLAYERS):
        agg = _sc_agg(x, edata, emb_p, zeros)
        mlp = _mlp_last if i == NLAYERS - 1 else _mlp_mid
        x = mlp(x, agg, W1[i], b1[i].reshape(1, H), W2[i],
                b2[i].reshape(1, H), g2, be2)
    return x
